# baseline (speedup 1.0000x reference)
import functools

import jax
import jax.numpy as jnp
from jax import lax
from jax.experimental import pallas as pl
from jax.experimental.pallas import tpu as pltpu
from jax.experimental.pallas import tpu_sc as plsc

_S = 64.0
_M3 = 0.35

_B = 1024
_C = 100000
_BR = 1000
_NB = _C // _BR

_NC = 2
_NS = 16
_L = 16
_NW = _NC * _NS
_PER_W = _B // _NW


def _scale_body(x_ref, o_ref):
    o_ref[...] = x_ref[...] * _S


def _scale_t(logits_t):
    return pl.pallas_call(
        _scale_body,
        grid=(_NB,),
        in_specs=[pl.BlockSpec((_BR, _B), lambda i: (i, 0))],
        out_specs=pl.BlockSpec((_BR, _B), lambda i: (i, 0)),
        out_shape=jax.ShapeDtypeStruct((_C, _B), jnp.float32),
    )(logits_t)


_sc_mesh = plsc.VectorSubcoreMesh(
    core_axis_name="c", subcore_axis_name="s", num_cores=_NC, num_subcores=_NS
)


@functools.partial(
    pl.kernel,
    mesh=_sc_mesh,
    scratch_types=[
        pltpu.VMEM((_PER_W,), jnp.int32),
        pltpu.VMEM((_PER_W, _PER_W), jnp.float32),
        pltpu.SemaphoreType.DMA,
    ],
)
def _sc_fix_t(logits_t, out_buf, labels_hbm, lab_v, wins, sem):
    wid = lax.axis_index("s") * _NC + lax.axis_index("c")
    base = wid * _PER_W
    pltpu.sync_copy(labels_hbm.at[pl.ds(base, _PER_W)], lab_v)
    chunks = [lab_v[pl.ds(j * _L, _L)] for j in range(_PER_W // _L)]
    rows = [chunks[r // _L][r % _L] for r in range(_PER_W)]
    gathers = [
        pltpu.async_copy(
            logits_t.at[rows[r], pl.ds(base, _PER_W)], wins.at[r], sem
        )
        for r in range(_PER_W)
    ]
    for g in gathers:
        g.wait()
    for r in range(_PER_W):
        for j in range(_PER_W // _L):
            x = wins[r, pl.ds(j * _L, _L)]
            hit = chunks[j] == rows[r]
            wins[r, pl.ds(j * _L, _L)] = jnp.where(hit, (x - _M3) * _S, x * _S)
    scatters = [
        pltpu.async_copy(
            wins.at[r], out_buf.at[rows[r], pl.ds(base, _PER_W)], sem
        )
        for r in range(_PER_W)
    ]
    for s in scatters:
        s.wait()


@jax.jit
def _combined(logits, labels):
    logits_t = logits.T
    scaled_t = _scale_t(logits_t)
    buf = jax.new_ref(scaled_t)
    _sc_fix_t(logits_t, buf, labels)
    return jax.freeze(buf).T


def kernel(logits, labels):
    return _combined(logits, labels.astype(jnp.int32))


# speedup vs baseline: 1.0059x; 1.0059x over previous
import functools

import jax
import jax.numpy as jnp
from jax import lax
from jax.experimental import pallas as pl
from jax.experimental.pallas import tpu as pltpu
from jax.experimental.pallas import tpu_sc as plsc

_S = 64.0
_M3 = 0.35

_B = 1024
_C = 100000
_BR = 2000
_NB = _C // _BR

_NC = 2
_NS = 16
_L = 16
_NW = _NC * _NS
_PER_W = _B // _NW


def _scale_body(x_ref, o_ref):
    o_ref[...] = x_ref[...] * _S


def _scale_t(logits_t):
    return pl.pallas_call(
        _scale_body,
        grid=(_NB,),
        in_specs=[pl.BlockSpec((_BR, _B), lambda i: (i, 0))],
        out_specs=pl.BlockSpec((_BR, _B), lambda i: (i, 0)),
        out_shape=jax.ShapeDtypeStruct((_C, _B), jnp.float32),
    )(logits_t)


_sc_mesh = plsc.VectorSubcoreMesh(
    core_axis_name="c", subcore_axis_name="s", num_cores=_NC, num_subcores=_NS
)


@functools.partial(
    pl.kernel,
    mesh=_sc_mesh,
    out_type=jax.ShapeDtypeStruct((_B, _PER_W), jnp.float32),
    scratch_types=[
        pltpu.VMEM((_PER_W,), jnp.int32),
        pltpu.VMEM((_PER_W, _PER_W), jnp.float32),
        pltpu.SemaphoreType.DMA,
    ],
)
def _sc_prep(logits_t, labels_hbm, wins_hbm, lab_v, wins, sem):
    wid = lax.axis_index("s") * _NC + lax.axis_index("c")
    base = wid * _PER_W
    pltpu.sync_copy(labels_hbm.at[pl.ds(base, _PER_W)], lab_v)
    chunks = [lab_v[pl.ds(j * _L, _L)] for j in range(_PER_W // _L)]
    rows = [chunks[r // _L][r % _L] for r in range(_PER_W)]
    gathers = [
        pltpu.async_copy(
            logits_t.at[rows[r], pl.ds(base, _PER_W)], wins.at[r], sem
        )
        for r in range(_PER_W)
    ]
    for g in gathers:
        g.wait()
    for r in range(_PER_W):
        for j in range(_PER_W // _L):
            x = wins[r, pl.ds(j * _L, _L)]
            hit = chunks[j] == rows[r]
            wins[r, pl.ds(j * _L, _L)] = jnp.where(hit, (x - _M3) * _S, x * _S)
    pltpu.sync_copy(wins, wins_hbm.at[pl.ds(base, _PER_W)])


@functools.partial(
    pl.kernel,
    mesh=_sc_mesh,
    scratch_types=[
        pltpu.VMEM((_PER_W,), jnp.int32),
        pltpu.VMEM((_PER_W, _PER_W), jnp.float32),
        pltpu.SemaphoreType.DMA,
    ],
)
def _sc_scatter(wins_hbm, out_buf, labels_hbm, lab_v, wins, sem):
    wid = lax.axis_index("s") * _NC + lax.axis_index("c")
    base = wid * _PER_W
    pltpu.sync_copy(labels_hbm.at[pl.ds(base, _PER_W)], lab_v)
    pltpu.sync_copy(wins_hbm.at[pl.ds(base, _PER_W)], wins)
    chunks = [lab_v[pl.ds(j * _L, _L)] for j in range(_PER_W // _L)]
    rows = [chunks[r // _L][r % _L] for r in range(_PER_W)]
    scatters = [
        pltpu.async_copy(
            wins.at[r], out_buf.at[rows[r], pl.ds(base, _PER_W)], sem
        )
        for r in range(_PER_W)
    ]
    for s in scatters:
        s.wait()


@jax.jit
def _combined(logits, labels):
    logits_t = logits.T
    wins = _sc_prep(logits_t, labels)
    scaled_t = _scale_t(logits_t)
    buf = jax.new_ref(scaled_t)
    _sc_scatter(wins, buf, labels)
    return jax.freeze(buf).T


def kernel(logits, labels):
    return _combined(logits, labels.astype(jnp.int32))


# speedup vs baseline: 1.0099x; 1.0040x over previous
import functools

import jax
import jax.numpy as jnp
from jax import lax
from jax.experimental import pallas as pl
from jax.experimental.pallas import tpu as pltpu
from jax.experimental.pallas import tpu_sc as plsc

_S = 64.0
_M3 = 0.35

_B = 1024
_C = 100000
_BR = 2000
_NB = _C // _BR

_NC = 2
_NS = 16
_L = 16
_NW = _NC * _NS
_PER_W = _B // _NW


def _scale_body(x_ref, o_ref):
    o_ref[...] = x_ref[...] * _S


def _scale_t(logits_t):
    return pl.pallas_call(
        _scale_body,
        grid=(_NB,),
        in_specs=[pl.BlockSpec((_BR, _B), lambda i: (i, 0))],
        out_specs=pl.BlockSpec((_BR, _B), lambda i: (i, 0)),
        out_shape=jax.ShapeDtypeStruct((_C, _B), jnp.float32),
    )(logits_t)


_sc_mesh = plsc.VectorSubcoreMesh(
    core_axis_name="c", subcore_axis_name="s", num_cores=_NC, num_subcores=_NS
)


@functools.partial(
    pl.kernel,
    mesh=_sc_mesh,
    scratch_types=[
        pltpu.VMEM((_PER_W,), jnp.int32),
        pltpu.VMEM((_PER_W, _PER_W), jnp.float32),
        pltpu.SemaphoreType.DMA,
    ],
)
def _sc_fix_t(logits_t, out_buf, labels_hbm, lab_v, wins, sem):
    wid = lax.axis_index("s") * _NC + lax.axis_index("c")
    base = wid * _PER_W
    pltpu.sync_copy(labels_hbm.at[pl.ds(base, _PER_W)], lab_v)
    chunks = [lab_v[pl.ds(j * _L, _L)] for j in range(_PER_W // _L)]
    rows = [chunks[r // _L][r % _L] for r in range(_PER_W)]
    gathers = [
        pltpu.async_copy(
            logits_t.at[rows[r], pl.ds(base, _PER_W)], wins.at[r], sem
        )
        for r in range(_PER_W)
    ]
    for g in gathers:
        g.wait()
    for r in range(_PER_W):
        for j in range(_PER_W // _L):
            x = wins[r, pl.ds(j * _L, _L)]
            hit = chunks[j] == rows[r]
            wins[r, pl.ds(j * _L, _L)] = jnp.where(hit, (x - _M3) * _S, x * _S)
    scatters = [
        pltpu.async_copy(
            wins.at[r], out_buf.at[rows[r], pl.ds(base, _PER_W)], sem
        )
        for r in range(_PER_W)
    ]
    for s in scatters:
        s.wait()


@jax.jit
def _combined(logits, labels):
    logits_t = logits.T
    scaled_t = _scale_t(logits_t)
    buf = jax.new_ref(scaled_t)
    _sc_fix_t(logits_t, buf, labels)
    return jax.freeze(buf).T


def kernel(logits, labels):
    return _combined(logits, labels.astype(jnp.int32))
